# baseline (speedup 1.0000x reference)
import functools

import jax
import jax.numpy as jnp
from jax import lax
from jax.experimental import pallas as pl
from jax.experimental.pallas import tpu as pltpu
from jax.experimental.pallas import tpu_sc as plsc

_B, _N, _K = 8, 2048, 4
_EPS = 1e-5
_TQ = 2048
_NKW = 4
_CHUNK = _N // _NKW
_NK = _N * _K


def _bf16r(x):
    return x.astype(jnp.bfloat16).astype(jnp.float32)


def _knn_body(keys_ref, idx_ref):
    keys16 = keys_ref[0, :, 0:2].astype(jnp.bfloat16)
    p = lax.dot_general(keys16, keys16, (((1,), (1,)), ((), ())),
                        preferred_element_type=jnp.float32)
    xm = keys_ref[0, :, 0:1]
    ym = keys_ref[0, :, 1:2]
    xxm = xm * xm + ym * ym
    xxq = jnp.transpose(xxm, (1, 0))
    dist = (xxm - 2.0 * p) + xxq
    iota = lax.broadcasted_iota(jnp.int32, (_N, _TQ), 0).astype(jnp.float32)
    for k in range(_K):
        mn = jnp.min(dist, axis=0, keepdims=True)
        cand = jnp.where(dist == mn, iota, jnp.float32(_N))
        sel = jnp.min(cand, axis=0, keepdims=True)
        idx_ref[0, k:k + 1, :] = sel.astype(jnp.int32)
        if k + 1 < _K:
            dist = jnp.where(iota == sel, jnp.float32(jnp.inf), dist)


def _knn_topk(xyz):
    return pl.pallas_call(
        _knn_body,
        grid=(_B,),
        in_specs=[pl.BlockSpec((1, _N, 3), lambda b: (b, 0, 0))],
        out_specs=pl.BlockSpec((1, _K, _TQ), lambda b: (b, 0, 0)),
        out_shape=jax.ShapeDtypeStruct((_B, _K, _N), jnp.int32),
    )(xyz)


def _sc_gather_body(xyzf_hbm, idxf_hbm, out_hbm, xyz_v, idx_v, out_v):
    wid = lax.axis_index("s") * 2 + lax.axis_index("c")
    b = wid // _NKW
    n0 = (wid % _NKW) * _CHUNK
    pltpu.sync_copy(xyzf_hbm.at[pl.ds(b * (3 * _N), 3 * _N)], xyz_v)
    pltpu.sync_copy(idxf_hbm.at[pl.ds(b * _NK, _NK)], idx_v)
    lanes = lax.iota(jnp.int32, 16)

    def body(i, carry):
        n_lane = n0 + i * 16 + lanes
        f0 = n_lane * 12
        for c in range(3):
            acc = jnp.zeros((16,), jnp.float32)
            for k in range(_K):
                f = f0 + (3 * k + c)
                r = jnp.where(f >= 2 * _NK, jnp.int32(2),
                              jnp.where(f >= _NK, jnp.int32(1), jnp.int32(0)))
                s = f - r * _NK
                sp = jnp.bitwise_and(s, 3) * _N + lax.shift_right_logical(s, 2)
                j = plsc.load_gather(idx_v, [sp])
                val = plsc.load_gather(xyz_v, [j * 3 + r])
                acc = acc + val
            xv = plsc.load_gather(xyz_v, [n_lane * 3 + c])
            feat = xv - acc * 0.25
            pos = (c * _CHUNK + i * 16) + lanes
            plsc.store_scatter(out_v, [pos], feat)
        return carry

    lax.fori_loop(0, _CHUNK // 16, body, 0)
    for c in range(3):
        pltpu.sync_copy(out_v.at[pl.ds(c * _CHUNK, _CHUNK)],
                        out_hbm.at[pl.ds(b * (3 * _N) + c * _N + n0, _CHUNK)])


def _sc_gather(xyzf, idxf):
    mesh = plsc.VectorSubcoreMesh(core_axis_name="c", subcore_axis_name="s",
                                  num_cores=2, num_subcores=16)
    out = pl.kernel(
        _sc_gather_body,
        out_type=jax.ShapeDtypeStruct((_B * 3 * _N,), jnp.float32),
        mesh=mesh,
        compiler_params=pltpu.CompilerParams(use_tc_tiling_on_sc=False,
                                             needs_layout_passes=False),
        scratch_types=[
            pltpu.VMEM((3 * _N,), jnp.float32),
            pltpu.VMEM((_NK,), jnp.int32),
            pltpu.VMEM((3 * _CHUNK,), jnp.float32),
        ],
    )(xyzf, idxf)
    return out.reshape(_B, 3, _N)


def _tail_body(featT_ref, xyz_ref, W_ref, bvec_ref, bnw_ref, bnb_ref, out_ref):
    f = [_bf16r(featT_ref[:, c, :]) for c in range(3)]
    trans = []
    ssum = None
    for cp in range(3):
        t = (f[0] * _bf16r(W_ref[cp, 0]) + f[1] * _bf16r(W_ref[cp, 1])
             + f[2] * _bf16r(W_ref[cp, 2])) + bvec_ref[cp]
        trans.append(t)
        s = jnp.sum(t, axis=0, keepdims=True)
        ssum = s if ssum is None else ssum + s
    mean = ssum / 24.0
    vsum = None
    for cp in range(3):
        d = trans[cp] - mean
        v = jnp.sum(d * d, axis=0, keepdims=True)
        vsum = v if vsum is None else vsum + v
    var = vsum / 24.0
    denom = jnp.sqrt(var + _EPS)
    rows = []
    for cp in range(3):
        t = (trans[cp] - mean) / denom
        t = t * bnw_ref[0:1, :] + bnb_ref[0:1, :]
        t = jnp.where(t >= 0, t, 0.1 * t)
        rows.append(t)
    for b in range(_B):
        tb = jnp.concatenate([rows[cp][b:b + 1, :] for cp in range(3)], axis=0)
        out_ref[b, :, :] = xyz_ref[b, :, :] + jnp.transpose(tb, (1, 0))


def _tail(featT, xyz, W, bvec, bn_w, bn_b):
    TN = 2048
    return pl.pallas_call(
        _tail_body,
        grid=(_N // TN,),
        in_specs=[
            pl.BlockSpec((_B, 3, TN), lambda t: (0, 0, t)),
            pl.BlockSpec((_B, TN, 3), lambda t: (0, t, 0)),
            pl.BlockSpec(memory_space=pltpu.SMEM),
            pl.BlockSpec(memory_space=pltpu.SMEM),
            pl.BlockSpec((1, TN), lambda t: (0, t)),
            pl.BlockSpec((1, TN), lambda t: (0, t)),
        ],
        out_specs=pl.BlockSpec((_B, TN, 3), lambda t: (0, t, 0)),
        out_shape=jax.ShapeDtypeStruct((_B, _N, 3), jnp.float32),
    )(featT, xyz, W, bvec, bn_w.reshape(1, _N), bn_b.reshape(1, _N))


def kernel(xyz, W, b, bn_w, bn_b):
    idxKN = _knn_topk(xyz)
    featT = _sc_gather(xyz.reshape(_B * 3 * _N), idxKN.reshape(_B * _NK))
    return _tail(featT, xyz, W, b, bn_w, bn_b)


# speedup vs baseline: 1.0061x; 1.0061x over previous
import jax
import jax.numpy as jnp
from jax import lax
from jax.experimental import pallas as pl
from jax.experimental.pallas import tpu as pltpu
from jax.experimental.pallas import tpu_sc as plsc

_B, _N, _K = 8, 2048, 4
_EPS = 1e-5
_TQ = 2048
_NKW = 4
_CHUNK = _N // _NKW
_NK = _N * _K


def _bf16r(x):
    return x.astype(jnp.bfloat16).astype(jnp.float32)


def _knn_body(keys_ref, idx_ref):
    keys16 = keys_ref[0, :, 0:2].astype(jnp.bfloat16)
    p = lax.dot_general(keys16, keys16, (((1,), (1,)), ((), ())),
                        preferred_element_type=jnp.float32)
    xm = keys_ref[0, :, 0:1]
    ym = keys_ref[0, :, 1:2]
    xxm = xm * xm + ym * ym
    xxq = jnp.transpose(xxm, (1, 0))
    dist = (xxm - 2.0 * p) + xxq
    iota = lax.broadcasted_iota(jnp.int32, (_N, _TQ), 0).astype(jnp.float32)
    for k in range(_K):
        mn = jnp.min(dist, axis=0, keepdims=True)
        cand = jnp.where(dist == mn, iota, jnp.float32(_N))
        sel = jnp.min(cand, axis=0, keepdims=True)
        idx_ref[0, k:k + 1, :] = sel.astype(jnp.int32)
        if k + 1 < _K:
            dist = jnp.where(iota == sel, jnp.float32(jnp.inf), dist)


def _knn_topk(xyz):
    return pl.pallas_call(
        _knn_body,
        grid=(_B,),
        in_specs=[pl.BlockSpec((1, _N, 3), lambda b: (b, 0, 0))],
        out_specs=pl.BlockSpec((1, _K, _TQ), lambda b: (b, 0, 0)),
        out_shape=jax.ShapeDtypeStruct((_B, _K, _N), jnp.int32),
    )(xyz)


def _sc_gather_body(xyzf_hbm, idxf_hbm, out_hbm, xyz_v, idx_v, out_v):
    wid = lax.axis_index("s") * 2 + lax.axis_index("c")
    b = wid // _NKW
    n0 = (wid % _NKW) * _CHUNK
    pltpu.sync_copy(xyzf_hbm.at[pl.ds(b * (3 * _N), 3 * _N)], xyz_v)
    pltpu.sync_copy(idxf_hbm.at[pl.ds(b * _NK, _NK)], idx_v)
    lanes = lax.iota(jnp.int32, 16)

    def body(i, carry):
        n_lane = n0 + i * 16 + lanes
        f0 = n_lane * 12
        for c in range(3):
            acc = jnp.zeros((16,), jnp.float32)
            for k in range(_K):
                f = f0 + (3 * k + c)
                r = jnp.where(f >= 2 * _NK, jnp.int32(2),
                              jnp.where(f >= _NK, jnp.int32(1), jnp.int32(0)))
                s = f - r * _NK
                sp = jnp.bitwise_and(s, 3) * _N + lax.shift_right_logical(s, 2)
                j = plsc.load_gather(idx_v, [sp])
                val = plsc.load_gather(xyz_v, [j * 3 + r])
                acc = acc + val
            xv = plsc.load_gather(xyz_v, [n_lane * 3 + c])
            feat = xv - acc * 0.25
            pos = (c * _CHUNK + i * 16) + lanes
            plsc.store_scatter(out_v, [pos], feat)
        return carry

    lax.fori_loop(0, _CHUNK // 16, body, 0)
    for c in range(3):
        pltpu.sync_copy(out_v.at[pl.ds(c * _CHUNK, _CHUNK)],
                        out_hbm.at[pl.ds(b * (3 * _N) + c * _N + n0, _CHUNK)])


def _sc_gather(xyzf, idxf):
    mesh = plsc.VectorSubcoreMesh(core_axis_name="c", subcore_axis_name="s",
                                  num_cores=2, num_subcores=16)
    out = pl.kernel(
        _sc_gather_body,
        out_type=jax.ShapeDtypeStruct((_B * 3 * _N,), jnp.float32),
        mesh=mesh,
        compiler_params=pltpu.CompilerParams(use_tc_tiling_on_sc=False,
                                             needs_layout_passes=False),
        scratch_types=[
            pltpu.VMEM((3 * _N,), jnp.float32),
            pltpu.VMEM((_NK,), jnp.int32),
            pltpu.VMEM((3 * _CHUNK,), jnp.float32),
        ],
    )(xyzf, idxf)
    return out.reshape(_B, 3, _N)


def _tail_body(featT_ref, xyz_ref, W_ref, bvec_ref, bnw_ref, bnb_ref, out_ref):
    f = [_bf16r(featT_ref[:, c, :]) for c in range(3)]
    trans = []
    ssum = None
    for cp in range(3):
        t = (f[0] * _bf16r(W_ref[cp, 0]) + f[1] * _bf16r(W_ref[cp, 1])
             + f[2] * _bf16r(W_ref[cp, 2])) + bvec_ref[cp]
        trans.append(t)
        s = jnp.sum(t, axis=0, keepdims=True)
        ssum = s if ssum is None else ssum + s
    mean = ssum / 24.0
    vsum = None
    for cp in range(3):
        d = trans[cp] - mean
        v = jnp.sum(d * d, axis=0, keepdims=True)
        vsum = v if vsum is None else vsum + v
    var = vsum / 24.0
    denom = jnp.sqrt(var + _EPS)
    rows = []
    for cp in range(3):
        t = (trans[cp] - mean) / denom
        t = t * bnw_ref[0:1, :] + bnb_ref[0:1, :]
        t = jnp.where(t >= 0, t, 0.1 * t)
        rows.append(t)
    for b in range(_B):
        tb = jnp.concatenate([rows[cp][b:b + 1, :] for cp in range(3)], axis=0)
        out_ref[b, :, :] = xyz_ref[b, :, :] + jnp.transpose(tb, (1, 0))


def _tail(featT, xyz, W, bvec, bn_w, bn_b):
    TN = 512
    return pl.pallas_call(
        _tail_body,
        grid=(_N // TN,),
        in_specs=[
            pl.BlockSpec((_B, 3, TN), lambda t: (0, 0, t)),
            pl.BlockSpec((_B, TN, 3), lambda t: (0, t, 0)),
            pl.BlockSpec(memory_space=pltpu.SMEM),
            pl.BlockSpec(memory_space=pltpu.SMEM),
            pl.BlockSpec((1, TN), lambda t: (0, t)),
            pl.BlockSpec((1, TN), lambda t: (0, t)),
        ],
        out_specs=pl.BlockSpec((_B, TN, 3), lambda t: (0, t, 0)),
        out_shape=jax.ShapeDtypeStruct((_B, _N, 3), jnp.float32),
    )(featT, xyz, W, bvec, bn_w.reshape(1, _N), bn_b.reshape(1, _N))


def kernel(xyz, W, b, bn_w, bn_b):
    idxKN = _knn_topk(xyz)
    featT = _sc_gather(xyz.reshape(_B * 3 * _N), idxKN.reshape(_B * _NK))
    return _tail(featT, xyz, W, b, bn_w, bn_b)
